# baseline (speedup 1.0000x reference)
import functools
import math

import jax
import jax.numpy as jnp
from jax.experimental import pallas as pl
from jax.experimental.pallas import tpu as pltpu
from jax.experimental.pallas import tpu_sc as plsc

N = 4096
D = 512
DH = 4 * D
RADIUS = 2
SIM_BETA = 1.0
GRID_MAX = 64

BR = 512
BQ = 256
BA = 512
BK = 256
NR = N // BR
NQ = N // BQ
NQA = N // BA
NKB = N // BK
NEG = -1e30


def _ln(x, w, b, eps=1e-5):
    mu = jnp.mean(x, axis=-1, keepdims=True)
    var = jnp.mean((x - mu) ** 2, axis=-1, keepdims=True)
    return (x - mu) / jnp.sqrt(var + eps) * w + b


def _dot_t(a, b):
    return jax.lax.dot_general(a, b, (((1,), (1,)), ((), ())),
                               preferred_element_type=jnp.float32)


def _gelu_tanh(x):
    c = math.sqrt(2.0 / math.pi)
    return 0.5 * x * (1.0 + jnp.tanh(c * (x + 0.044715 * x * x * x)))


@functools.lru_cache(maxsize=None)
def _make_sc_scatter(B, Dm, Da):
    info = plsc.get_sparse_core_info()
    nw = info.num_cores * info.num_subcores
    b_per_w = B // nw
    mesh = plsc.VectorSubcoreMesh(core_axis_name="c", subcore_axis_name="s")

    @functools.partial(
        pl.kernel, mesh=mesh,
        out_type=[jax.ShapeDtypeStruct((B, Dm), jnp.float32),
                  jax.ShapeDtypeStruct((B, Da), jnp.int32)],
        scratch_types=[
            pltpu.VMEM((b_per_w,), jnp.int32),
            pltpu.VMEM((b_per_w, Dm), jnp.float32),
            pltpu.VMEM((b_per_w, Da), jnp.int32),
            pltpu.SemaphoreType.DMA,
            pltpu.SemaphoreType.DMA,
        ],
    )
    def k(table_hbm, aux_hbm, idx_hbm, out_hbm, outaux_hbm,
          idx_v, rows_v, aux_v, sem, sem2):
        wid = jax.lax.axis_index("s") * info.num_cores + jax.lax.axis_index("c")
        base = wid * b_per_w
        sl = pl.ds(base, b_per_w)
        pltpu.sync_copy(idx_hbm.at[sl], idx_v)
        pltpu.sync_copy(table_hbm.at[sl], rows_v)
        pltpu.sync_copy(aux_hbm.at[sl], aux_v)
        c1 = pltpu.async_copy(rows_v, out_hbm.at[idx_v], sem)
        c2 = pltpu.async_copy(aux_v, outaux_hbm.at[idx_v], sem2)
        c1.wait()
        c2.wait()

    return k


@functools.lru_cache(maxsize=None)
def _make_sc_gather(B, Dm):
    info = plsc.get_sparse_core_info()
    nw = info.num_cores * info.num_subcores
    b_per_w = B // nw
    mesh = plsc.VectorSubcoreMesh(core_axis_name="c", subcore_axis_name="s")

    @functools.partial(
        pl.kernel, mesh=mesh,
        out_type=jax.ShapeDtypeStruct((B, Dm), jnp.float32),
        scratch_types=[
            pltpu.VMEM((b_per_w,), jnp.int32),
            pltpu.VMEM((b_per_w, Dm), jnp.float32),
            pltpu.SemaphoreType.DMA,
        ],
    )
    def k(table_hbm, idx_hbm, out_hbm, idx_v, rows_v, sem):
        wid = jax.lax.axis_index("s") * info.num_cores + jax.lax.axis_index("c")
        base = wid * b_per_w
        pltpu.sync_copy(idx_hbm.at[pl.ds(base, b_per_w)], idx_v)
        pltpu.async_copy(table_hbm.at[idx_v], rows_v, sem).wait()
        pltpu.sync_copy(rows_v, out_hbm.at[pl.ds(base, b_per_w)])

    return k


def _permute_rows(table, idx):
    return _make_sc_gather(table.shape[0], table.shape[1])(table, idx)


def _rank_kernel(gyc_ref, rank_ref):
    f32 = jnp.float32
    r_i = jax.lax.broadcasted_iota(jnp.int32, (BQ, BQ), 0)
    c_i = jax.lax.broadcasted_iota(jnp.int32, (BQ, BQ), 1)
    ltri = (c_i < r_i).astype(f32)
    lane = jax.lax.broadcasted_iota(jnp.int32, (BQ, GRID_MAX), 1)

    colsum = jnp.zeros((1, GRID_MAX), f32)
    ohs, intras = [], []
    for b in range(NQ):
        gyb = gyc_ref[pl.ds(b * BQ, BQ), :]
        oh = (gyb == lane).astype(f32)
        cum_in = jax.lax.dot_general(ltri, oh, (((1,), (0,)), ((), ())),
                                     preferred_element_type=f32)
        intras.append(jnp.sum(oh * (cum_in + colsum), axis=1,
                              keepdims=True))
        ohs.append(oh)
        colsum = colsum + jnp.sum(oh, axis=0, keepdims=True)

    bi = jax.lax.broadcasted_iota(jnp.int32, (GRID_MAX, GRID_MAX), 0)
    bj = jax.lax.broadcasted_iota(jnp.int32, (GRID_MAX, GRID_MAX), 1)
    excl = (bi < bj).astype(f32)
    cumt = jax.lax.dot_general(colsum, excl, (((1,), (0,)), ((), ())),
                               preferred_element_type=f32)
    for b in range(NQ):
        offs = jnp.sum(ohs[b] * cumt, axis=1, keepdims=True)
        rank_ref[pl.ds(b * BQ, BQ), :] = (intras[b] + offs).astype(jnp.int32)


def _block_kernel(spans_ref,
                  x1_ref, x3_ref, gxq_ref, gyq_ref, gxr_ref, gyr_ref,
                  wq_ref, bq_ref, wk_ref, bk_ref, wv_ref, bv_ref,
                  ln1w_ref, ln1b_ref,
                  wo_ref, bo_ref, ln2w_ref, ln2b_ref,
                  w1_ref, b1_ref, w2_ref, b2_ref,
                  o_ref,
                  qc_ref, kc_ref, v_ref, att_ref,
                  m_ref, l_ref, acc_ref, any_ref):
    step = pl.program_id(0)
    bf16 = jnp.bfloat16

    @pl.when(step < NR)
    def _qkv():
        b = step
        x = x1_ref[...]
        h = _ln(x, ln1w_ref[...], ln1b_ref[...])
        nrm = jnp.sqrt(jnp.sum(h * h, axis=-1, keepdims=True))
        hn = h / jnp.maximum(nrm, 1e-8)
        scale = 1.0 / math.sqrt(D)
        hb = h.astype(bf16)
        q = _dot_t(hb, wq_ref[...]) + bq_ref[...]
        k = _dot_t(hb, wk_ref[...]) + bk_ref[...]
        v = _dot_t(hb, wv_ref[...]) + bv_ref[...]
        rs = pl.ds(b * BR, BR)
        qc_ref[rs, :D] = (q * scale).astype(bf16)
        qc_ref[rs, D:] = (hn * SIM_BETA).astype(bf16)
        kc_ref[rs, :D] = k.astype(bf16)
        kc_ref[rs, D:] = hn.astype(bf16)
        v_ref[rs, :] = v.astype(bf16)

    @pl.when((step >= NR) & (step < NR + NQA))
    def _attn():
        i = step - NR
        lo = spans_ref[2 * i]
        hi = spans_ref[2 * i + 1]

        m_ref[...] = jnp.full_like(m_ref, NEG)
        l_ref[...] = jnp.zeros_like(l_ref)
        acc_ref[...] = jnp.zeros_like(acc_ref)
        any_ref[...] = jnp.zeros_like(any_ref)

        qs = pl.ds(i * BA, BA)
        qc = qc_ref[qs, :]
        gxq = gxq_ref[qs, :]
        gyq = gyq_ref[qs, :]
        rid = jax.lax.broadcasted_iota(jnp.int32, (BA, BK), 0) + i * BA

        def body(jb, _):
            kc = kc_ref[pl.ds(jb * BK, BK), :]
            scores = _dot_t(qc, kc)
            gxk = gxr_ref[pl.ds(jb, 1)][0]
            gyk = gyr_ref[pl.ds(jb, 1)][0]
            dx = jnp.abs(gxq - gxk)
            dy = jnp.abs(gyq - gyk)
            cid = jax.lax.broadcasted_iota(jnp.int32, (BA, BK), 1) + jb * BK
            mask = (dx <= RADIUS) & (dy <= RADIUS) & (rid != cid)

            logits = jnp.where(mask, scores, NEG)
            m_prev = m_ref[...]
            m_new = jnp.maximum(m_prev,
                                jnp.max(logits, axis=1, keepdims=True))
            alpha = jnp.exp(m_prev - m_new)
            p = jnp.exp(logits - m_new)
            p = jnp.where(mask, p, 0.0)
            l_ref[...] = l_ref[...] * alpha + jnp.sum(p, axis=1,
                                                      keepdims=True)
            acc_ref[...] = acc_ref[...] * alpha + jax.lax.dot_general(
                p.astype(bf16), v_ref[pl.ds(jb * BK, BK), :],
                (((1,), (0,)), ((), ())),
                preferred_element_type=jnp.float32)
            m_ref[...] = m_new
            any_ref[...] = jnp.maximum(
                any_ref[...],
                jnp.max(mask.astype(jnp.float32), axis=1, keepdims=True))
            return 0

        jax.lax.fori_loop(lo, hi, body, 0)

        l = jnp.maximum(l_ref[...], 1e-30)
        out = acc_ref[...] / l
        vq = v_ref[qs, :].astype(jnp.float32)
        att_ref[qs, :] = jnp.where(any_ref[...] > 0.0, out,
                                   vq).astype(bf16)

    @pl.when(step >= NR + NQA)
    def _ffn():
        b = step - (NR + NQA)
        att = att_ref[pl.ds(b * BR, BR), :]
        x2 = x3_ref[...] + _dot_t(att, wo_ref[...]) + bo_ref[...]
        h2 = _ln(x2, ln2w_ref[...], ln2b_ref[...])
        a1 = _dot_t(h2.astype(bf16), w1_ref[...]) + b1_ref[...]
        g = _gelu_tanh(a1)
        ffn = _dot_t(g.astype(bf16), w2_ref[...]) + b2_ref[...]
        o_ref[...] = x2 + ffn


def kernel(x, grid, Wq, bq, Wk, bk, Wv, bv, Wo, bo, ln1_w, ln1_b,
           ln2_w, ln2_b, W1, b1, W2, b2):
    f32 = jnp.float32
    i32 = jnp.int32
    bf16 = jnp.bfloat16

    gx = grid[:, 0].astype(i32)
    gy = grid[:, 1].astype(i32)
    cell = gy * GRID_MAX + gx
    gyc = gy.reshape(N, 1)
    cellpad = jnp.broadcast_to(cell[:, None], (N, 128))

    rank2d = pl.pallas_call(
        _rank_kernel,
        grid=(1,),
        in_specs=[pl.BlockSpec((N, 1), lambda i: (0, 0))],
        out_specs=pl.BlockSpec((N, 1), lambda i: (0, 0)),
        out_shape=jax.ShapeDtypeStruct((N, 1), i32),
    )(gyc)
    rank = rank2d.reshape(N)

    x_s, cells_s8 = _make_sc_scatter(N, D, 128)(x, cellpad, rank)

    cs = cells_s8[:, 0]
    gxs = (cs % GRID_MAX).astype(f32)
    gys_i = cs // GRID_MAX
    gys = gys_i.astype(f32)

    r_lo = gys_i.reshape(NQA, BA)[:, 0]
    r_hi = gys_i.reshape(NQA, BA)[:, -1]
    lo = jnp.searchsorted(gys_i, r_lo - RADIUS, side="left")
    hi = jnp.searchsorted(gys_i, r_hi + RADIUS + 1, side="left")
    lob = jnp.clip(lo // BK, 0, NKB - 1).astype(i32)
    hib = jnp.clip((hi + BK - 1) // BK, lob + 1, NKB).astype(i32)
    spans = jnp.stack([lob, hib], axis=1).reshape(-1)

    gxq = gxs.reshape(N, 1)
    gyq = gys.reshape(N, 1)
    gxr = gxs.reshape(NKB, 1, BK)
    gyr = gys.reshape(NKB, 1, BK)

    Wq_b, Wk_b, Wv_b = Wq.astype(bf16), Wk.astype(bf16), Wv.astype(bf16)
    Wo_b, W1_b, W2_b = Wo.astype(bf16), W1.astype(bf16), W2.astype(bf16)

    full = lambda *s: pl.BlockSpec(s, lambda i, sp: (0,) * len(s))
    NSTEPS = NR + NQA + NR

    def x1_map(i, sp):
        return (jnp.minimum(i, NR - 1), 0)

    def x3_map(i, sp):
        return (jnp.clip(i - (NR + NQA), 0, NR - 1), 0)

    def o_map(i, sp):
        return (jnp.clip(i - (NR + NQA), 0, NR - 1), 0)

    out_s = pl.pallas_call(
        _block_kernel,
        grid_spec=pltpu.PrefetchScalarGridSpec(
            num_scalar_prefetch=1,
            grid=(NSTEPS,),
            in_specs=[
                pl.BlockSpec((BR, D), x1_map),
                pl.BlockSpec((BR, D), x3_map),
                full(N, 1), full(N, 1),
                full(NKB, 1, BK), full(NKB, 1, BK),
                full(D, D), full(D), full(D, D), full(D), full(D, D),
                full(D), full(D), full(D),
                full(D, D), full(D), full(D), full(D),
                full(DH, D), full(DH), full(D, DH), full(D),
            ],
            out_specs=pl.BlockSpec((BR, D), o_map),
            scratch_shapes=[
                pltpu.VMEM((N, 2 * D), bf16),
                pltpu.VMEM((N, 2 * D), bf16),
                pltpu.VMEM((N, D), bf16),
                pltpu.VMEM((N, D), bf16),
                pltpu.VMEM((BA, 1), f32),
                pltpu.VMEM((BA, 1), f32),
                pltpu.VMEM((BA, D), f32),
                pltpu.VMEM((BA, 1), f32),
            ],
        ),
        out_shape=jax.ShapeDtypeStruct((N, D), f32),
        compiler_params=pltpu.CompilerParams(
            dimension_semantics=("arbitrary",)),
    )(spans, x_s, x_s, gxq, gyq, gxr, gyr,
      Wq_b, bq, Wk_b, bk, Wv_b, bv, ln1_w, ln1_b,
      Wo_b, bo, ln2_w, ln2_b, W1_b, b1, W2_b, b2)

    return _permute_rows(out_s, rank)


# speedup vs baseline: 1.0900x; 1.0900x over previous
import functools
import math

import jax
import jax.numpy as jnp
from jax.experimental import pallas as pl
from jax.experimental.pallas import tpu as pltpu
from jax.experimental.pallas import tpu_sc as plsc

N = 4096
D = 512
DH = 4 * D
RADIUS = 2
SIM_BETA = 1.0
GRID_MAX = 64

BR = 512
BQ = 256
BA = 256
BK = 512
NR = N // BR
NQ = N // BQ
NQA = N // BA
NKB = N // BK
NEG = -1e30


def _ln(x, w, b, eps=1e-5):
    mu = jnp.mean(x, axis=-1, keepdims=True)
    var = jnp.mean((x - mu) ** 2, axis=-1, keepdims=True)
    return (x - mu) / jnp.sqrt(var + eps) * w + b


def _dot_t(a, b):
    return jax.lax.dot_general(a, b, (((1,), (1,)), ((), ())),
                               preferred_element_type=jnp.float32)


def _gelu_tanh(x):
    c = math.sqrt(2.0 / math.pi)
    return 0.5 * x * (1.0 + jnp.tanh(c * (x + 0.044715 * x * x * x)))


@functools.lru_cache(maxsize=None)
def _make_sc_scatter(B, Dm, Da):
    info = plsc.get_sparse_core_info()
    nw = info.num_cores * info.num_subcores
    b_per_w = B // nw
    mesh = plsc.VectorSubcoreMesh(core_axis_name="c", subcore_axis_name="s")

    @functools.partial(
        pl.kernel, mesh=mesh,
        out_type=[jax.ShapeDtypeStruct((B, Dm), jnp.float32),
                  jax.ShapeDtypeStruct((B, Da), jnp.int32)],
        scratch_types=[
            pltpu.VMEM((b_per_w,), jnp.int32),
            pltpu.VMEM((b_per_w, Dm), jnp.float32),
            pltpu.VMEM((b_per_w, Da), jnp.int32),
            pltpu.SemaphoreType.DMA,
            pltpu.SemaphoreType.DMA,
        ],
    )
    def k(table_hbm, aux_hbm, idx_hbm, out_hbm, outaux_hbm,
          idx_v, rows_v, aux_v, sem, sem2):
        wid = jax.lax.axis_index("s") * info.num_cores + jax.lax.axis_index("c")
        base = wid * b_per_w
        sl = pl.ds(base, b_per_w)
        pltpu.sync_copy(idx_hbm.at[sl], idx_v)
        pltpu.sync_copy(table_hbm.at[sl], rows_v)
        pltpu.sync_copy(aux_hbm.at[sl], aux_v)
        c1 = pltpu.async_copy(rows_v, out_hbm.at[idx_v], sem)
        c2 = pltpu.async_copy(aux_v, outaux_hbm.at[idx_v], sem2)
        c1.wait()
        c2.wait()

    return k


@functools.lru_cache(maxsize=None)
def _make_sc_gather(B, Dm):
    info = plsc.get_sparse_core_info()
    nw = info.num_cores * info.num_subcores
    b_per_w = B // nw
    mesh = plsc.VectorSubcoreMesh(core_axis_name="c", subcore_axis_name="s")

    @functools.partial(
        pl.kernel, mesh=mesh,
        out_type=jax.ShapeDtypeStruct((B, Dm), jnp.float32),
        scratch_types=[
            pltpu.VMEM((b_per_w,), jnp.int32),
            pltpu.VMEM((b_per_w, Dm), jnp.float32),
            pltpu.SemaphoreType.DMA,
        ],
    )
    def k(table_hbm, idx_hbm, out_hbm, idx_v, rows_v, sem):
        wid = jax.lax.axis_index("s") * info.num_cores + jax.lax.axis_index("c")
        base = wid * b_per_w
        pltpu.sync_copy(idx_hbm.at[pl.ds(base, b_per_w)], idx_v)
        pltpu.async_copy(table_hbm.at[idx_v], rows_v, sem).wait()
        pltpu.sync_copy(rows_v, out_hbm.at[pl.ds(base, b_per_w)])

    return k


def _permute_rows(table, idx):
    return _make_sc_gather(table.shape[0], table.shape[1])(table, idx)


def _rank_kernel(gyc_ref, rank_ref):
    f32 = jnp.float32
    r_i = jax.lax.broadcasted_iota(jnp.int32, (BQ, BQ), 0)
    c_i = jax.lax.broadcasted_iota(jnp.int32, (BQ, BQ), 1)
    ltri = (c_i < r_i).astype(f32)
    lane = jax.lax.broadcasted_iota(jnp.int32, (BQ, GRID_MAX), 1)

    colsum = jnp.zeros((1, GRID_MAX), f32)
    ohs, intras = [], []
    for b in range(NQ):
        gyb = gyc_ref[pl.ds(b * BQ, BQ), :]
        oh = (gyb == lane).astype(f32)
        cum_in = jax.lax.dot_general(ltri, oh, (((1,), (0,)), ((), ())),
                                     preferred_element_type=f32)
        intras.append(jnp.sum(oh * (cum_in + colsum), axis=1,
                              keepdims=True))
        ohs.append(oh)
        colsum = colsum + jnp.sum(oh, axis=0, keepdims=True)

    bi = jax.lax.broadcasted_iota(jnp.int32, (GRID_MAX, GRID_MAX), 0)
    bj = jax.lax.broadcasted_iota(jnp.int32, (GRID_MAX, GRID_MAX), 1)
    excl = (bi < bj).astype(f32)
    cumt = jax.lax.dot_general(colsum, excl, (((1,), (0,)), ((), ())),
                               preferred_element_type=f32)
    for b in range(NQ):
        offs = jnp.sum(ohs[b] * cumt, axis=1, keepdims=True)
        rank_ref[pl.ds(b * BQ, BQ), :] = (intras[b] + offs).astype(jnp.int32)


def _block_kernel(spans_ref,
                  x1_ref, x3_ref, gxq_ref, gyq_ref, gxr_ref, gyr_ref,
                  wq_ref, bq_ref, wk_ref, bk_ref, wv_ref, bv_ref,
                  ln1w_ref, ln1b_ref,
                  wo_ref, bo_ref, ln2w_ref, ln2b_ref,
                  w1_ref, b1_ref, w2_ref, b2_ref,
                  o_ref,
                  qc_ref, kc_ref, v_ref, att_ref,
                  m_ref, l_ref, acc_ref, any_ref):
    step = pl.program_id(0)
    bf16 = jnp.bfloat16

    @pl.when(step < NR)
    def _qkv():
        b = step
        x = x1_ref[...]
        h = _ln(x, ln1w_ref[...], ln1b_ref[...])
        nrm = jnp.sqrt(jnp.sum(h * h, axis=-1, keepdims=True))
        hn = h / jnp.maximum(nrm, 1e-8)
        scale = 1.0 / math.sqrt(D)
        hb = h.astype(bf16)
        q = _dot_t(hb, wq_ref[...]) + bq_ref[...]
        k = _dot_t(hb, wk_ref[...]) + bk_ref[...]
        v = _dot_t(hb, wv_ref[...]) + bv_ref[...]
        rs = pl.ds(b * BR, BR)
        qc_ref[rs, :D] = (q * scale).astype(bf16)
        qc_ref[rs, D:] = (hn * SIM_BETA).astype(bf16)
        kc_ref[rs, :D] = k.astype(bf16)
        kc_ref[rs, D:] = hn.astype(bf16)
        v_ref[rs, :] = v.astype(bf16)

    @pl.when((step >= NR) & (step < NR + NQA))
    def _attn():
        i = step - NR
        lo = spans_ref[2 * i]
        hi = spans_ref[2 * i + 1]

        m_ref[...] = jnp.full_like(m_ref, NEG)
        l_ref[...] = jnp.zeros_like(l_ref)
        acc_ref[...] = jnp.zeros_like(acc_ref)
        any_ref[...] = jnp.zeros_like(any_ref)

        qs = pl.ds(i * BA, BA)
        qc = qc_ref[qs, :]
        gxq = gxq_ref[qs, :]
        gyq = gyq_ref[qs, :]
        rid = jax.lax.broadcasted_iota(jnp.int32, (BA, BK), 0) + i * BA

        def body(jb, _):
            kc = kc_ref[pl.ds(jb * BK, BK), :]
            scores = _dot_t(qc, kc)
            gxk = gxr_ref[pl.ds(jb, 1)][0]
            gyk = gyr_ref[pl.ds(jb, 1)][0]
            dx = jnp.abs(gxq - gxk)
            dy = jnp.abs(gyq - gyk)
            cid = jax.lax.broadcasted_iota(jnp.int32, (BA, BK), 1) + jb * BK
            mask = (dx <= RADIUS) & (dy <= RADIUS) & (rid != cid)

            logits = jnp.where(mask, scores, NEG)
            m_prev = m_ref[...]
            m_new = jnp.maximum(m_prev,
                                jnp.max(logits, axis=1, keepdims=True))
            alpha = jnp.exp(m_prev - m_new)
            p = jnp.exp(logits - m_new)
            p = jnp.where(mask, p, 0.0)
            l_ref[...] = l_ref[...] * alpha + jnp.sum(p, axis=1,
                                                      keepdims=True)
            acc_ref[...] = acc_ref[...] * alpha + jax.lax.dot_general(
                p.astype(bf16), v_ref[pl.ds(jb * BK, BK), :],
                (((1,), (0,)), ((), ())),
                preferred_element_type=jnp.float32)
            m_ref[...] = m_new
            any_ref[...] = jnp.maximum(
                any_ref[...],
                jnp.max(mask.astype(jnp.float32), axis=1, keepdims=True))
            return 0

        jax.lax.fori_loop(lo, hi, body, 0)

        l = jnp.maximum(l_ref[...], 1e-30)
        out = acc_ref[...] / l
        vq = v_ref[qs, :].astype(jnp.float32)
        att_ref[qs, :] = jnp.where(any_ref[...] > 0.0, out,
                                   vq).astype(bf16)

    @pl.when(step >= NR + NQA)
    def _ffn():
        b = step - (NR + NQA)
        att = att_ref[pl.ds(b * BR, BR), :]
        x2 = x3_ref[...] + _dot_t(att, wo_ref[...]) + bo_ref[...]
        h2 = _ln(x2, ln2w_ref[...], ln2b_ref[...])
        a1 = _dot_t(h2.astype(bf16), w1_ref[...]) + b1_ref[...]
        g = _gelu_tanh(a1)
        ffn = _dot_t(g.astype(bf16), w2_ref[...]) + b2_ref[...]
        o_ref[...] = x2 + ffn


def kernel(x, grid, Wq, bq, Wk, bk, Wv, bv, Wo, bo, ln1_w, ln1_b,
           ln2_w, ln2_b, W1, b1, W2, b2):
    f32 = jnp.float32
    i32 = jnp.int32
    bf16 = jnp.bfloat16

    gx = grid[:, 0].astype(i32)
    gy = grid[:, 1].astype(i32)
    cell = gy * GRID_MAX + gx
    gyc = gy.reshape(N, 1)
    cellpad = jnp.broadcast_to(cell[:, None], (N, 128))

    rank2d = pl.pallas_call(
        _rank_kernel,
        grid=(1,),
        in_specs=[pl.BlockSpec((N, 1), lambda i: (0, 0))],
        out_specs=pl.BlockSpec((N, 1), lambda i: (0, 0)),
        out_shape=jax.ShapeDtypeStruct((N, 1), i32),
    )(gyc)
    rank = rank2d.reshape(N)

    x_s, cells_s8 = _make_sc_scatter(N, D, 128)(x, cellpad, rank)

    cs = cells_s8[:, 0]
    gxs = (cs % GRID_MAX).astype(f32)
    gys_i = cs // GRID_MAX
    gys = gys_i.astype(f32)

    r_lo = gys_i.reshape(NQA, BA)[:, 0]
    r_hi = gys_i.reshape(NQA, BA)[:, -1]
    lo = jnp.searchsorted(gys_i, r_lo - RADIUS, side="left")
    hi = jnp.searchsorted(gys_i, r_hi + RADIUS + 1, side="left")
    lob = jnp.clip(lo // BK, 0, NKB - 1).astype(i32)
    hib = jnp.clip((hi + BK - 1) // BK, lob + 1, NKB).astype(i32)
    spans = jnp.stack([lob, hib], axis=1).reshape(-1)

    gxq = gxs.reshape(N, 1)
    gyq = gys.reshape(N, 1)
    gxr = gxs.reshape(NKB, 1, BK)
    gyr = gys.reshape(NKB, 1, BK)

    Wq_b, Wk_b, Wv_b = Wq.astype(bf16), Wk.astype(bf16), Wv.astype(bf16)
    Wo_b, W1_b, W2_b = Wo.astype(bf16), W1.astype(bf16), W2.astype(bf16)

    full = lambda *s: pl.BlockSpec(s, lambda i, sp: (0,) * len(s))
    NSTEPS = NR + NQA + NR

    def x1_map(i, sp):
        return (jnp.minimum(i, NR - 1), 0)

    def x3_map(i, sp):
        return (jnp.clip(i - (NR + NQA), 0, NR - 1), 0)

    def o_map(i, sp):
        return (jnp.clip(i - (NR + NQA), 0, NR - 1), 0)

    out_s = pl.pallas_call(
        _block_kernel,
        grid_spec=pltpu.PrefetchScalarGridSpec(
            num_scalar_prefetch=1,
            grid=(NSTEPS,),
            in_specs=[
                pl.BlockSpec((BR, D), x1_map),
                pl.BlockSpec((BR, D), x3_map),
                full(N, 1), full(N, 1),
                full(NKB, 1, BK), full(NKB, 1, BK),
                full(D, D), full(D), full(D, D), full(D), full(D, D),
                full(D), full(D), full(D),
                full(D, D), full(D), full(D), full(D),
                full(DH, D), full(DH), full(D, DH), full(D),
            ],
            out_specs=pl.BlockSpec((BR, D), o_map),
            scratch_shapes=[
                pltpu.VMEM((N, 2 * D), bf16),
                pltpu.VMEM((N, 2 * D), bf16),
                pltpu.VMEM((N, D), bf16),
                pltpu.VMEM((N, D), bf16),
                pltpu.VMEM((BA, 1), f32),
                pltpu.VMEM((BA, 1), f32),
                pltpu.VMEM((BA, D), f32),
                pltpu.VMEM((BA, 1), f32),
            ],
        ),
        out_shape=jax.ShapeDtypeStruct((N, D), f32),
        compiler_params=pltpu.CompilerParams(
            dimension_semantics=("arbitrary",)),
    )(spans, x_s, x_s, gxq, gyq, gxr, gyr,
      Wq_b, bq, Wk_b, bk, Wv_b, bv, ln1_w, ln1_b,
      Wo_b, bo, ln2_w, ln2_b, W1_b, b1, W2_b, b2)

    return _permute_rows(out_s, rank)
